# baseline (speedup 1.0000x reference)
import jax
import jax.numpy as jnp
from jax.experimental import pallas as pl
from jax.experimental.pallas import tpu as pltpu

_BLKW = 128
_BLKT = 2048


def _fused_kernel(mask_ref, w_ref, x_ref, b_ref, out_ref, wmix_ref,
                  bmix_ref):
    s = pl.program_id(0)
    e_dim, blkw, _ = w_ref.shape
    nkw = wmix_ref.shape[0] // blkw

    @pl.when(s == 0)
    def _bias():
        bmix = b_ref[0:1, :] * mask_ref[0]
        for e in range(1, e_dim):
            bmix += b_ref[e:e + 1, :] * mask_ref[e]
        bmix_ref[...] = bmix

    @pl.when(s < nkw)
    def _mix():
        acc = w_ref[0] * mask_ref[0]
        for e in range(1, e_dim):
            acc += w_ref[e] * mask_ref[e]
        wmix_ref[pl.ds(s * blkw, blkw), :] = acc

    @pl.when(s >= nkw)
    def _matmul():
        acc = jnp.dot(x_ref[...], wmix_ref[...],
                      preferred_element_type=jnp.float32)
        out_ref[...] = acc + bmix_ref[...]


def kernel(x, mask, W, b):
    t, d = x.shape
    e = W.shape[0]
    nkw = d // _BLKW
    nt = t // _BLKT

    out = pl.pallas_call(
        _fused_kernel,
        grid=(nkw + nt,),
        in_specs=[
            pl.BlockSpec(memory_space=pltpu.MemorySpace.SMEM),
            pl.BlockSpec((e, _BLKW, d),
                         lambda s: (0, jnp.minimum(s, nkw - 1), 0)),
            pl.BlockSpec((_BLKT, d),
                         lambda s: (jnp.maximum(s - nkw, 0), 0)),
            pl.BlockSpec((e, d), lambda s: (0, 0)),
        ],
        out_specs=pl.BlockSpec((_BLKT, d),
                               lambda s: (jnp.maximum(s - nkw, 0), 0)),
        out_shape=jax.ShapeDtypeStruct((t, d), jnp.float32),
        scratch_shapes=[pltpu.VMEM((d, d), jnp.float32),
                        pltpu.VMEM((1, d), jnp.float32)],
    )(mask, W, x, b)

    return (out, mask)


# speedup vs baseline: 1.0235x; 1.0235x over previous
import jax
import jax.numpy as jnp
from jax.experimental import pallas as pl
from jax.experimental.pallas import tpu as pltpu

_BLKW = 256
_BLKT = 2048


def _fused_kernel(mask_ref, w_ref, x_ref, b_ref, out_ref, wmix_ref,
                  bmix_ref):
    s = pl.program_id(0)
    e_dim, blkw, _ = w_ref.shape
    nkw = wmix_ref.shape[0] // blkw

    @pl.when(s == 0)
    def _bias():
        bmix = b_ref[0:1, :] * mask_ref[0]
        for e in range(1, e_dim):
            bmix += b_ref[e:e + 1, :] * mask_ref[e]
        bmix_ref[...] = bmix

    @pl.when(s < nkw)
    def _mix():
        acc = w_ref[0] * mask_ref[0]
        for e in range(1, e_dim):
            acc += w_ref[e] * mask_ref[e]
        wmix_ref[pl.ds(s * blkw, blkw), :] = acc

    @pl.when(s >= nkw)
    def _matmul():
        acc = jnp.dot(x_ref[...], wmix_ref[...],
                      preferred_element_type=jnp.float32)
        out_ref[...] = acc + bmix_ref[...]


def kernel(x, mask, W, b):
    t, d = x.shape
    e = W.shape[0]
    nkw = d // _BLKW
    nt = t // _BLKT

    out = pl.pallas_call(
        _fused_kernel,
        grid=(nkw + nt,),
        in_specs=[
            pl.BlockSpec(memory_space=pltpu.MemorySpace.SMEM),
            pl.BlockSpec((e, _BLKW, d),
                         lambda s: (0, jnp.minimum(s, nkw - 1), 0)),
            pl.BlockSpec((_BLKT, d),
                         lambda s: (jnp.maximum(s - nkw, 0), 0)),
            pl.BlockSpec((e, d), lambda s: (0, 0)),
        ],
        out_specs=pl.BlockSpec((_BLKT, d),
                               lambda s: (jnp.maximum(s - nkw, 0), 0)),
        out_shape=jax.ShapeDtypeStruct((t, d), jnp.float32),
        scratch_shapes=[pltpu.VMEM((d, d), jnp.float32),
                        pltpu.VMEM((1, d), jnp.float32)],
    )(mask, W, x, b)

    return (out, mask)
